# Initial kernel scaffold; baseline (speedup 1.0000x reference)
import functools

import jax
import jax.numpy as jnp
from jax import lax
from jax.experimental import pallas as pl
from jax.experimental.pallas import tpu as pltpu
from jax.experimental.pallas import tpu_sc as plsc

B = 16384
F = 26
FACT = 16
L = 16
NC = 2
NS = 16
NW = NC * NS
RPW = B // NW
W = 64
NCHUNK = RPW // W


def kernel(x, emb_table, fc_table, lin_w, lin_b):
    x_flat = x.reshape(-1)
    fc_flat = fc_table.reshape(-1)
    w_vec = jnp.broadcast_to(lin_w.reshape(1), (L,)).astype(jnp.float32)
    b_vec = jnp.broadcast_to(lin_b.reshape(1), (L,)).astype(jnp.float32)

    mesh = plsc.VectorSubcoreMesh(core_axis_name="c", subcore_axis_name="s")

    @functools.partial(
        pl.kernel,
        out_type=jax.ShapeDtypeStruct((B,), jnp.float32),
        mesh=mesh,
        scratch_types=[
            pltpu.VMEM((W * F,), jnp.int32),
            pltpu.VMEM((W * F, FACT), jnp.float32),
            pltpu.VMEM((W * F + L,), jnp.float32),
            pltpu.VMEM((RPW,), jnp.float32),
            pltpu.VMEM((RPW,), jnp.float32),
            pltpu.VMEM((L,), jnp.float32),
            pltpu.VMEM((L,), jnp.float32),
            pltpu.SemaphoreType.DMA,
            pltpu.SemaphoreType.DMA,
        ],
    )
    def sc_kernel(x_hbm, emb_hbm, fc_hbm, w_hbm, b_hbm, out_hbm,
                  idx_v, rows_v, fcv, sv, fv, wv, bv, sem_e, sem_f):
        wid = lax.axis_index("s") * NC + lax.axis_index("c")
        base = wid * RPW

        pltpu.sync_copy(w_hbm, wv)
        pltpu.sync_copy(b_hbm, bv)

        lanes = lax.iota(jnp.int32, (L,), 0)
        tail_mask = lanes < (F - L)

        @pl.loop(0, NCHUNK)
        def _chunk(c):
            cbase = (base + c * W) * F
            pltpu.sync_copy(x_hbm.at[pl.ds(cbase, W * F)], idx_v)
            e_cp = pltpu.async_copy(emb_hbm.at[idx_v], rows_v, sem_e)
            f_cp = pltpu.async_copy(fc_hbm.at[idx_v], fcv.at[pl.ds(0, W * F)],
                                    sem_f)
            e_cp.wait()
            f_cp.wait()

            @pl.loop(0, W)
            def _row(r):
                rb = r * F
                v = rows_v[rb, :]
                acc = v
                accsq = v * v
                for f in range(1, F):
                    v = rows_v[rb + f, :]
                    acc = acc + v
                    accsq = accsq + v * v
                inter = jnp.sum(acc * acc - accsq)

                f1 = fcv[pl.ds(rb, L)]
                f2 = fcv[pl.ds(rb + L, L)]
                f2 = jnp.where(tail_mask, f2, 0.0)
                fcs = jnp.sum(f1 + f2)

                sv[c * W + r] = inter
                fv[c * W + r] = fcs

        wvec = wv[...]
        bvec = bv[...]

        @pl.loop(0, RPW, step=L)
        def _final(i):
            z = 0.5 * sv[pl.ds(i, L)] + wvec * fv[pl.ds(i, L)] + bvec
            sv[pl.ds(i, L)] = 1.0 / (1.0 + jnp.exp(-z))

        pltpu.sync_copy(sv, out_hbm.at[pl.ds(base, RPW)])

    out = sc_kernel(x_flat, emb_table, fc_flat, w_vec, b_vec)
    return out.reshape(B, 1)


# speedup vs baseline: 1.3460x; 1.3460x over previous
import dataclasses
import functools

import jax
import jax.numpy as jnp
from jax import lax
from jax.experimental import pallas as pl
from jax.experimental.pallas import tpu as pltpu
from jax.experimental.pallas import tpu_sc as plsc

B = 16384
F = 26
FACT = 16
L = 16
NC = 2
NS = 16
NW = NC * NS
RPW = B // NW
W = 64
NCHUNK = RPW // W


def kernel(x, emb_table, fc_table, lin_w, lin_b):
    x_flat = x.reshape(-1)
    fc_flat = fc_table.reshape(-1)
    w_vec = jnp.broadcast_to(lin_w.reshape(1), (L,)).astype(jnp.float32)
    b_vec = jnp.broadcast_to(lin_b.reshape(1), (L,)).astype(jnp.float32)

    mesh = plsc.VectorSubcoreMesh(core_axis_name="c", subcore_axis_name="s")
    cp = pltpu.CompilerParams()
    if "needs_layout_passes" in pltpu.CompilerParams.__dataclass_fields__:
        cp = dataclasses.replace(cp, needs_layout_passes=False)
    if "use_tc_tiling_on_sc" in pltpu.CompilerParams.__dataclass_fields__:
        cp = dataclasses.replace(cp, use_tc_tiling_on_sc=False)

    @functools.partial(
        pl.kernel,
        out_type=jax.ShapeDtypeStruct((B,), jnp.float32),
        mesh=mesh,
        compiler_params=cp,
        scratch_types=[
            pltpu.VMEM((W * F,), jnp.int32),
            pltpu.VMEM((W * F, FACT), jnp.float32),
            pltpu.VMEM((W * F + L,), jnp.float32),
            pltpu.VMEM((RPW,), jnp.float32),
            pltpu.VMEM((L,), jnp.float32),
            pltpu.VMEM((L,), jnp.float32),
            pltpu.SemaphoreType.DMA,
            pltpu.SemaphoreType.DMA,
        ],
    )
    def sc_kernel(x_hbm, emb_hbm, fc_hbm, w_hbm, b_hbm, out_hbm,
                  idx_v, rows_v, fcv, sv, wv, bv, sem_e, sem_f):
        wid = lax.axis_index("s") * NC + lax.axis_index("c")
        base = wid * RPW

        pltpu.sync_copy(w_hbm, wv)
        pltpu.sync_copy(b_hbm, bv)

        lanes = lax.iota(jnp.int32, L)
        tail_mask = lanes < (F - L)
        wvec = wv[...]
        bvec = bv[...]

        @pl.loop(0, NCHUNK)
        def _chunk(c):
            cbase = (base + c * W) * F
            pltpu.sync_copy(x_hbm.at[pl.ds(cbase, W * F)], idx_v)
            e_cp = pltpu.async_copy(emb_hbm.at[idx_v], rows_v, sem_e)
            f_cp = pltpu.async_copy(fc_hbm.at[idx_v], fcv.at[pl.ds(0, W * F)],
                                    sem_f)
            e_cp.wait()
            f_cp.wait()

            @pl.loop(0, W // L)
            def _group(g):
                zacc = jnp.zeros((L,), jnp.float32)
                for j in range(L):
                    rb = (g * L + j) * F
                    v = rows_v[rb, :]
                    acc = v
                    accsq = v * v
                    for f in range(1, F):
                        v = rows_v[rb + f, :]
                        acc = acc + v
                        accsq = accsq + v * v
                    inter = jnp.sum(acc * acc - accsq)

                    f1 = fcv[pl.ds(rb, L)]
                    f2 = fcv[pl.ds(rb + L, L)]
                    f2 = jnp.where(tail_mask, f2, 0.0)
                    fcs = jnp.sum(f1 + f2)

                    z = 0.5 * inter + wvec * fcs + bvec
                    res = 1.0 / (1.0 + jnp.exp(-z))
                    zacc = jnp.where(lanes == j, res, zacc)
                sv[pl.ds(c * W + g * L, L)] = zacc

        pltpu.sync_copy(sv, out_hbm.at[pl.ds(base, RPW)])

    out = sc_kernel(x_flat, emb_table, fc_flat, w_vec, b_vec)
    return out.reshape(B, 1)


# speedup vs baseline: 1.3748x; 1.0214x over previous
import dataclasses
import functools

import jax
import jax.numpy as jnp
from jax import lax
from jax.experimental import pallas as pl
from jax.experimental.pallas import tpu as pltpu
from jax.experimental.pallas import tpu_sc as plsc

B = 16384
F = 26
FACT = 16
L = 16
NC = 2
NS = 16
NW = NC * NS
RPW = B // NW
W = 64
NCHUNK = RPW // W


def kernel(x, emb_table, fc_table, lin_w, lin_b):
    xf = x.T.reshape(-1)
    fc_flat = fc_table.T.reshape(-1)
    w_vec = jnp.broadcast_to(lin_w.reshape(1), (L,)).astype(jnp.float32)
    b_vec = jnp.broadcast_to(lin_b.reshape(1), (L,)).astype(jnp.float32)

    mesh = plsc.VectorSubcoreMesh(core_axis_name="c", subcore_axis_name="s")
    cp = pltpu.CompilerParams()
    if "needs_layout_passes" in pltpu.CompilerParams.__dataclass_fields__:
        cp = dataclasses.replace(cp, needs_layout_passes=False)
    if "use_tc_tiling_on_sc" in pltpu.CompilerParams.__dataclass_fields__:
        cp = dataclasses.replace(cp, use_tc_tiling_on_sc=False)

    @functools.partial(
        pl.kernel,
        out_type=jax.ShapeDtypeStruct((B,), jnp.float32),
        mesh=mesh,
        compiler_params=cp,
        scratch_types=[
            pltpu.VMEM((W * F,), jnp.int32),
            pltpu.VMEM((W * F,), jnp.int32),
            pltpu.VMEM((W * F, FACT), jnp.float32),
            pltpu.VMEM((W * F, FACT), jnp.float32),
            pltpu.VMEM((W * F,), jnp.float32),
            pltpu.VMEM((W * F,), jnp.float32),
            pltpu.VMEM((RPW,), jnp.float32),
            pltpu.VMEM((L,), jnp.float32),
            pltpu.VMEM((L,), jnp.float32),
            pltpu.SemaphoreType.DMA,
            pltpu.SemaphoreType.DMA,
            pltpu.SemaphoreType.DMA,
            pltpu.SemaphoreType.DMA,
            pltpu.SemaphoreType.DMA,
            pltpu.SemaphoreType.DMA,
        ],
    )
    def sc_kernel(x_hbm, emb_hbm, fc_hbm, w_hbm, b_hbm, out_hbm,
                  idx0, idx1, rows0, rows1, fcv0, fcv1, sv, wv, bv,
                  si0, si1, se0, se1, sf0, sf1):
        wid = lax.axis_index("s") * NC + lax.axis_index("c")
        base = wid * RPW

        pltpu.sync_copy(w_hbm, wv)
        pltpu.sync_copy(b_hbm, bv)

        lanes = lax.iota(jnp.int32, L)
        wvec = wv[...]
        bvec = bv[...]

        bufs = ((idx0, rows0, fcv0, si0, se0, sf0),
                (idx1, rows1, fcv1, si1, se1, sf1))

        def idx_args(c, b):
            idx_v, _, _, si, _, _ = bufs[b]
            cb = base + c * W
            return [(x_hbm.at[pl.ds(f * B + cb, W)],
                     idx_v.at[pl.ds(f * W, W)], si) for f in range(F)]

        def issue_idx(c, b):
            for src, dst, sem in idx_args(c, b):
                pltpu.async_copy(src, dst, sem)

        def wait_idx(c, b):
            for src, dst, sem in idx_args(c, b):
                pltpu.make_async_copy(src, dst, sem).wait()

        def gather_args(b):
            idx_v, rows, fcv, _, se, sf = bufs[b]
            return ((emb_hbm.at[idx_v], rows, se),
                    (fc_hbm.at[idx_v], fcv, sf))

        def issue_gather(b):
            for src, dst, sem in gather_args(b):
                pltpu.async_copy(src, dst, sem)

        def wait_gather(b):
            for src, dst, sem in gather_args(b):
                pltpu.make_async_copy(src, dst, sem).wait()

        def compute(c, b):
            _, rows, fcv, _, _, _ = bufs[b]

            @pl.loop(0, W // L)
            def _group(g):
                fcs = fcv[pl.ds(g * L, L)]
                for f in range(1, F):
                    fcs = fcs + fcv[pl.ds(f * W + g * L, L)]

                zacc = jnp.zeros((L,), jnp.float32)
                for j in range(L):
                    rb = g * L + j
                    v = rows[rb, :]
                    acc = v
                    accsq = v * v
                    for f in range(1, F):
                        v = rows[f * W + rb, :]
                        acc = acc + v
                        accsq = accsq + v * v
                    inter = jnp.sum(acc * acc - accsq)
                    zacc = jnp.where(lanes == j, inter, zacc)

                z = 0.5 * zacc + wvec * fcs + bvec
                sv[pl.ds(c * W + g * L, L)] = 1.0 / (1.0 + jnp.exp(-z))

        issue_idx(0, 0)
        issue_idx(1, 1)
        wait_idx(0, 0)
        issue_gather(0)
        for c in range(NCHUNK):
            b = c % 2
            wait_gather(b)
            if c + 2 < NCHUNK:
                issue_idx(c + 2, b)
            if c + 1 < NCHUNK:
                wait_idx(c + 1, 1 - b)
                issue_gather(1 - b)
            compute(c, b)

        pltpu.sync_copy(sv, out_hbm.at[pl.ds(base, RPW)])

    out = sc_kernel(xf, emb_table, fc_flat, w_vec, b_vec)
    return out.reshape(B, 1)
